# baseline (speedup 1.0000x reference)
import functools

import jax
import jax.numpy as jnp
from jax import lax
from jax.experimental import pallas as pl
from jax.experimental.pallas import tpu as pltpu
from jax.experimental.pallas import tpu_sc as plsc

NC = 2
NS = 16
L = 16
NW = NC * NS

EPS = 1e-5
CHUNK = 512
IDX_GRP = 128
NGRP = CHUNK // IDX_GRP


def _rsqrt(x):
    i = lax.bitcast_convert_type(x, jnp.int32)
    i = jnp.int32(0x5F3759DF) - lax.shift_right_logical(i, 1)
    y = lax.bitcast_convert_type(i, jnp.float32)
    half = jnp.float32(0.5) * x
    for _ in range(3):
        y = y * (jnp.float32(1.5) - half * y * y)
    return y


def _make_kernel(N, V, D):
    assert D == 4 * L
    per_w = N // NW
    n_chunks = per_w // CHUNK
    assert per_w % CHUNK == 0
    mesh = plsc.VectorSubcoreMesh(
        core_axis_name="c", subcore_axis_name="s", num_cores=NC, num_subcores=NS
    )

    @functools.partial(
        pl.kernel,
        out_type=jax.ShapeDtypeStruct((N, D), jnp.float32),
        mesh=mesh,
        scratch_types=[
            pltpu.VMEM((CHUNK,), jnp.int32),
            pltpu.VMEM((CHUNK, D), jnp.float32),
            pltpu.VMEM((2, D), jnp.float32),
            pltpu.SemaphoreType.DMA,
        ],
        compiler_params=pltpu.CompilerParams(use_tc_tiling_on_sc=False),
    )
    def k(idx_hbm, table_hbm, gb_hbm, out_hbm, idx_v, rows_v, gb_v, sem):
        wid = lax.axis_index("s") * NC + lax.axis_index("c")
        row0 = wid * per_w

        pltpu.sync_copy(gb_hbm, gb_v)
        gam = [gb_v[0, pl.ds(kk * L, L)] for kk in range(4)]
        bet = [gb_v[1, pl.ds(kk * L, L)] for kk in range(4)]
        inv_d = jnp.float32(1.0 / D)
        ix = lax.iota(jnp.int32, L)
        perms = [lax.bitwise_xor(ix, jnp.int32(d)) for d in (8, 4, 2, 1)]

        dnums = lax.GatherDimensionNumbers(
            offset_dims=(), collapsed_slice_dims=(0,), start_index_map=(0,)
        )

        def lane_sum(v):
            for p in perms:
                v = v + lax.gather(
                    v, p[:, None], dnums, (1,),
                    unique_indices=True,
                    mode=lax.GatherScatterMode.PROMISE_IN_BOUNDS,
                )
            return v

        def chunk_body(g, carry):
            rbase = row0 + g * CHUNK
            pltpu.sync_copy(idx_hbm.at[pl.ds(rbase, CHUNK)], idx_v)
            handles = []
            for j in range(NGRP):
                handles.append(
                    pltpu.async_copy(
                        table_hbm.at[idx_v.at[pl.ds(j * IDX_GRP, IDX_GRP)]],
                        rows_v.at[pl.ds(j * IDX_GRP, IDX_GRP)],
                        sem,
                    )
                )
            for h in handles:
                h.wait()

            def row_body(r, c2):
                a = [rows_v[r, pl.ds(kk * L, L)] for kk in range(4)]
                s = (a[0] + a[1]) + (a[2] + a[3])
                q = (a[0] * a[0] + a[1] * a[1]) + (a[2] * a[2] + a[3] * a[3])
                mean = lane_sum(s) * inv_d
                var = lane_sum(q) * inv_d - mean * mean
                rstd = _rsqrt(var + jnp.float32(EPS))
                for kk in range(4):
                    rows_v[r, pl.ds(kk * L, L)] = (
                        (a[kk] - mean) * rstd * gam[kk] + bet[kk]
                    )
                return c2

            lax.fori_loop(0, CHUNK, row_body, 0)
            pltpu.sync_copy(rows_v, out_hbm.at[pl.ds(rbase, CHUNK)])
            return carry

        lax.fori_loop(0, n_chunks, chunk_body, 0)

    return k


def kernel(x, table, gamma, beta):
    B, S = x.shape
    V, D = table.shape
    N = B * S
    idx = x.reshape(N).astype(jnp.int32)
    gb = jnp.stack([gamma, beta]).astype(jnp.float32)
    out = _make_kernel(N, V, D)(idx, table, gb)
    return out.reshape(B, S, D)


# speedup vs baseline: 1.4739x; 1.4739x over previous
import functools

import jax
import jax.numpy as jnp
from jax import lax
from jax.experimental import pallas as pl
from jax.experimental.pallas import tpu as pltpu
from jax.experimental.pallas import tpu_sc as plsc

NC = 2
NS = 16
L = 16
NW = NC * NS

EPS = 1e-5
CHUNK = 512
IDX_GRP = 128
NGRP = CHUNK // IDX_GRP


def _rsqrt(x):
    i = lax.bitcast_convert_type(x, jnp.int32)
    i = jnp.int32(0x5F3759DF) - lax.shift_right_logical(i, 1)
    y = lax.bitcast_convert_type(i, jnp.float32)
    half = jnp.float32(0.5) * x
    for _ in range(3):
        y = y * (jnp.float32(1.5) - half * y * y)
    return y


def _make_kernel(N, V, D):
    assert D == 4 * L
    per_w = N // NW
    n_chunks = per_w // CHUNK
    assert per_w % CHUNK == 0
    mesh = plsc.VectorSubcoreMesh(
        core_axis_name="c", subcore_axis_name="s", num_cores=NC, num_subcores=NS
    )

    @functools.partial(
        pl.kernel,
        out_type=jax.ShapeDtypeStruct((N, D), jnp.float32),
        mesh=mesh,
        scratch_types=[
            pltpu.VMEM((2, CHUNK), jnp.int32),
            pltpu.VMEM((2, CHUNK, D), jnp.float32),
            pltpu.VMEM((2, D), jnp.float32),
            pltpu.SemaphoreType.DMA,
            pltpu.SemaphoreType.DMA,
        ],
        compiler_params=pltpu.CompilerParams(use_tc_tiling_on_sc=False),
    )
    def k(idx_hbm, table_hbm, gb_hbm, out_hbm, idx_v, rows_v, gb_v, sem_g, sem_o):
        wid = lax.axis_index("s") * NC + lax.axis_index("c")
        row0 = wid * per_w

        pltpu.sync_copy(gb_hbm, gb_v)
        gam = [gb_v[0, pl.ds(kk * L, L)] for kk in range(4)]
        bet = [gb_v[1, pl.ds(kk * L, L)] for kk in range(4)]
        inv_d = jnp.float32(1.0 / D)
        ix = lax.iota(jnp.int32, L)
        perms = [lax.bitwise_xor(ix, jnp.int32(d)) for d in (8, 4, 2, 1)]
        dnums = lax.GatherDimensionNumbers(
            offset_dims=(), collapsed_slice_dims=(0,), start_index_map=(0,)
        )

        def lane_sum(v):
            for p in perms:
                v = v + lax.gather(
                    v, p[:, None], dnums, (1,),
                    unique_indices=True,
                    mode=lax.GatherScatterMode.PROMISE_IN_BOUNDS,
                )
            return v

        def start_gathers(g, b):
            rbase = row0 + g * CHUNK
            pltpu.sync_copy(idx_hbm.at[pl.ds(rbase, CHUNK)], idx_v.at[b])
            for j in range(NGRP):
                pltpu.async_copy(
                    table_hbm.at[idx_v.at[b, pl.ds(j * IDX_GRP, IDX_GRP)]],
                    rows_v.at[b, pl.ds(j * IDX_GRP, IDX_GRP)],
                    sem_g,
                )

        def wait_rows(b, sem):
            pltpu.make_async_copy(
                out_hbm.at[pl.ds(0, CHUNK)], rows_v.at[b], sem
            ).wait()

        start_gathers(0, 0)

        def chunk_body(g, carry):
            b = g % 2
            wait_rows(b, sem_g)

            @pl.when(g + 1 < n_chunks)
            def _():
                @pl.when(g >= 1)
                def _():
                    wait_rows(1 - b, sem_o)
                start_gathers(g + 1, 1 - b)

            @plsc.parallel_loop(0, CHUNK, unroll=8)
            def row_body(r):
                a = [rows_v[b, r, pl.ds(kk * L, L)] for kk in range(4)]
                s = (a[0] + a[1]) + (a[2] + a[3])
                q = (a[0] * a[0] + a[1] * a[1]) + (a[2] * a[2] + a[3] * a[3])
                mean = lane_sum(s) * inv_d
                var = lane_sum(q) * inv_d - mean * mean
                rstd = _rsqrt(var + jnp.float32(EPS))
                for kk in range(4):
                    rows_v[b, r, pl.ds(kk * L, L)] = (
                        (a[kk] - mean) * rstd * gam[kk] + bet[kk]
                    )

            pltpu.async_copy(
                rows_v.at[b], out_hbm.at[pl.ds(row0 + g * CHUNK, CHUNK)], sem_o
            )
            return carry

        lax.fori_loop(0, n_chunks, chunk_body, 0)
        wait_rows(0, sem_o)
        wait_rows(1, sem_o)

    return k


def kernel(x, table, gamma, beta):
    B, S = x.shape
    V, D = table.shape
    N = B * S
    idx = x.reshape(N).astype(jnp.int32)
    gb = jnp.stack([gamma, beta]).astype(jnp.float32)
    out = _make_kernel(N, V, D)(idx, table, gb)
    return out.reshape(B, S, D)
